# baseline (speedup 1.0000x reference)
import functools

import jax
import jax.numpy as jnp
from jax import lax
from jax.experimental import pallas as pl
from jax.experimental.pallas import tpu as pltpu


def _b(x):
    return x.astype(jnp.bfloat16)


def _dot(a, b):
    return lax.dot_general(_b(a), _b(b), (((1,), (0,)), ((), ())),
                           preferred_element_type=jnp.float32)


def _dot_ta(a, b):
    return lax.dot_general(_b(a), _b(b), (((0,), (0,)), ((), ())),
                           preferred_element_type=jnp.float32)


def _dot_tb(a, b):
    return lax.dot_general(_b(a), _b(b), (((1,), (1,)), ((), ())),
                           preferred_element_type=jnp.float32)


def _signs(latent, grdpg, dtype):
    lane = lax.broadcasted_iota(jnp.int32, (1, latent), 1)
    return jnp.where(lane >= latent - grdpg, -1.0, 1.0).astype(dtype)


_N_CHUNKS = 4


def _enc_kernel(sidx_ref,
                x1_ref, x2_ref, adj_ref, wb1_ref, wm1_ref, wls1_ref,
                wb2_ref, wm2_ref, wls2_ref,
                n1_ref, n2_ref, n3_ref, zb_ref,
                adj_scr, x_scr, na_scr, nb_scr, z1b_scr,
                adj_sem, x_sem, na_sem, nb_sem,
                *, latent, grdpg, n_species, n1_size):
    i = pl.program_id(0)
    chunk = n1_size // _N_CHUNKS

    def _row_copies(x_hbm, na_hbm, nb_hbm):
        copies = []
        for k in range(_N_CHUNKS):
            c = pltpu.make_async_copy(
                adj_ref.at[pl.ds(k * chunk, chunk), :],
                adj_scr.at[pl.ds(k * chunk, chunk), :],
                adj_sem.at[k])
            c.start()
            copies.append(c)
        xc = pltpu.make_async_copy(x_hbm, x_scr, x_sem)
        xc.start()
        nac = pltpu.make_async_copy(na_hbm, na_scr, na_sem)
        nac.start()
        nbc = None
        if nb_hbm is not None:
            nbc = pltpu.make_async_copy(nb_hbm, nb_scr, nb_sem)
            nbc.start()
        return copies, xc, nac, nbc

    @pl.when(i == 0)
    def _encoder1():
        copies, xc, nac, nbc = _row_copies(x1_ref, n1_ref, n3_ref)
        xc.wait()
        p1 = _dot(x_scr[...], wb1_ref[...])
        h1 = jnp.zeros((n1_size, p1.shape[1]), jnp.float32)
        for k in range(_N_CHUNKS):
            copies[k].wait()
            h1 += _dot_ta(adj_scr[pl.ds(k * chunk, chunk), :],
                          p1[k * chunk:(k + 1) * chunk, :])
        h1 = jnp.maximum(h1, 0.0)
        adj = adj_scr[...]
        mean1 = _dot(adj, _b(_dot(h1, wm1_ref[...])))
        std1 = jnp.exp(_dot(adj, _b(_dot(h1, wls1_ref[...]))))
        nac.wait()
        nbc.wait()
        z1b = nb_scr[...] * std1 + mean1
        zb_ref[0, :, :latent] = _b(na_scr[...] * std1 + mean1)
        zb_ref[0, :, latent:] = _b(z1b)
        z1b_scr[...] = z1b
        rows = [z1b_scr[pl.ds(sidx_ref[j], 1), :] for j in range(n_species)]
        z3 = jnp.concatenate(rows, axis=0)
        zb_ref[0, :n_species, latent:] = _b(z3)

    @pl.when(i == 1)
    def _encoder2():
        copies, xc, nac, _ = _row_copies(x2_ref, n2_ref, None)
        xc.wait()
        p2 = _dot(x_scr[...], wb2_ref[...])
        h2_parts = []
        for k in range(_N_CHUNKS):
            copies[k].wait()
            h2_parts.append(_dot(adj_scr[pl.ds(k * chunk, chunk), :], p2))
        h2 = jnp.maximum(jnp.concatenate(h2_parts, axis=0), 0.0)
        adj = adj_scr[...]
        mean2 = _dot_ta(adj, _b(_dot(h2, wm2_ref[...])))
        std2 = jnp.exp(_dot_ta(adj, _b(_dot(h2, wls2_ref[...]))))
        nac.wait()
        z2s = _b((na_scr[...] * std2 + mean2)
                 * _signs(latent, grdpg, jnp.float32))
        zb_ref[0, :, :latent] = z2s
        zb_ref[0, :, latent:] = z2s


def _dec_kernel(z1t_ref, z2s_ref, z3b_ref,
                a_ref, a2_ref, z1o_ref, z2o_ref, z3_ref,
                *, latent, grdpg, sp_blk, half):
    i = pl.program_id(0)
    sg16 = _signs(latent, grdpg, jnp.bfloat16)
    z2s = z2s_ref[0]
    z1t = z1t_ref[0]
    z3h = z3b_ref[0, pl.ds(i * sp_blk, sp_blk), :]

    a_ref[...] = jax.nn.sigmoid(_dot_tb(z1t, z2s))
    a2_ref[...] = jax.nn.sigmoid(_dot_tb(z3h, z2s))
    z1o_ref[...] = z1t.astype(jnp.float32)
    z2o_ref[...] = (z2s_ref[0, pl.ds(i * half, half), :]
                    * sg16).astype(jnp.float32)
    z3_ref[...] = z3h.astype(jnp.float32)


@jax.jit
def kernel(w_base1, w_mean1, w_logstd1, w_base2, w_mean2, w_logstd2,
           X1, X2, adj, noise1, noise2, noise3, species_idx):
    N1, N2 = adj.shape
    latent = w_mean1.shape[1]
    n_species = species_idx.shape[0]

    d_in = X1.shape[1]
    hbm = pl.BlockSpec(memory_space=pl.ANY)
    enc_in = (X1, X2, adj, w_base1, w_mean1, w_logstd1,
              w_base2, w_mean2, w_logstd2, noise1, noise2, noise3)
    enc_specs = (
        [hbm, hbm, hbm]
        + [pl.BlockSpec(a.shape, lambda i, _sp, _n=a.ndim: (0,) * _n)
           for a in enc_in[3:9]]
        + [hbm, hbm, hbm]
    )

    zb = pl.pallas_call(
        functools.partial(_enc_kernel, latent=int(latent), grdpg=1,
                          n_species=int(n_species), n1_size=N1),
        grid_spec=pltpu.PrefetchScalarGridSpec(
            num_scalar_prefetch=1,
            grid=(2,),
            in_specs=enc_specs,
            out_specs=pl.BlockSpec((1, N1, 2 * latent),
                                   lambda i, _sp: (i, 0, 0)),
            scratch_shapes=[
                pltpu.VMEM((N1, N2), jnp.float32),
                pltpu.VMEM((N1, d_in), jnp.float32),
                pltpu.VMEM((N1, latent), jnp.float32),
                pltpu.VMEM((N1, latent), jnp.float32),
                pltpu.VMEM((N1, latent), jnp.float32),
                pltpu.SemaphoreType.DMA((_N_CHUNKS,)),
                pltpu.SemaphoreType.DMA,
                pltpu.SemaphoreType.DMA,
                pltpu.SemaphoreType.DMA,
            ],
        ),
        out_shape=jax.ShapeDtypeStruct((2, N1, 2 * latent), jnp.bfloat16),
        compiler_params=pltpu.CompilerParams(
            dimension_semantics=("parallel",)),
    )(species_idx, *enc_in)

    half = N1 // 2
    sp_blk = n_species // 2

    out_shapes = (
        jax.ShapeDtypeStruct((N1, N2), jnp.float32),
        jax.ShapeDtypeStruct((n_species, N2), jnp.float32),
        jax.ShapeDtypeStruct((N1, latent), jnp.float32),
        jax.ShapeDtypeStruct((N2, latent), jnp.float32),
        jax.ShapeDtypeStruct((n_species, latent), jnp.float32),
    )
    a_pred, a_pred2, z1o, z2o, z3 = pl.pallas_call(
        functools.partial(_dec_kernel, latent=int(latent), grdpg=1,
                          sp_blk=sp_blk, half=half),
        grid=(2,),
        in_specs=[
            pl.BlockSpec((1, half, latent), lambda i: (0, i, 0)),
            pl.BlockSpec((1, N2, latent), lambda i: (1, 0, 0)),
            pl.BlockSpec((1, n_species, latent), lambda i: (0, 0, 1)),
        ],
        out_specs=[
            pl.BlockSpec((half, N2), lambda i: (i, 0)),
            pl.BlockSpec((sp_blk, N2), lambda i: (i, 0)),
            pl.BlockSpec((half, latent), lambda i: (i, 0)),
            pl.BlockSpec((half, latent), lambda i: (i, 0)),
            pl.BlockSpec((sp_blk, latent), lambda i: (i, 0)),
        ],
        out_shape=out_shapes,
        compiler_params=pltpu.CompilerParams(
            dimension_semantics=("parallel",)),
    )(zb, zb, zb)

    return (a_pred, a_pred2, z1o, z2o, z3)


# speedup vs baseline: 1.4328x; 1.4328x over previous
import functools

import jax
import jax.numpy as jnp
from jax import lax
from jax.experimental import pallas as pl
from jax.experimental.pallas import tpu as pltpu


def _b(x):
    return x.astype(jnp.bfloat16)


def _dot(a, b):
    return lax.dot_general(_b(a), _b(b), (((1,), (0,)), ((), ())),
                           preferred_element_type=jnp.float32)


def _dot_ta(a, b):
    return lax.dot_general(_b(a), _b(b), (((0,), (0,)), ((), ())),
                           preferred_element_type=jnp.float32)


def _dot_tb(a, b):
    return lax.dot_general(_b(a), _b(b), (((1,), (1,)), ((), ())),
                           preferred_element_type=jnp.float32)


def _signs(latent, grdpg, dtype):
    lane = lax.broadcasted_iota(jnp.int32, (1, latent), 1)
    return jnp.where(lane >= latent - grdpg, -1.0, 1.0).astype(dtype)


def _enc_kernel(sidx_ref,
                x1_ref, x2_ref, adj_ref, wb1_ref, wm1_ref, wls1_ref,
                wb2_ref, wm2_ref, wls2_ref,
                n1_ref, n2_ref, n3_ref, zb_ref, z1b_scr,
                *, latent, grdpg, n_species):
    i = pl.program_id(0)

    @pl.when(i == 0)
    def _encoder1():
        adj = adj_ref[...]
        p1 = _dot(x1_ref[...], wb1_ref[...])
        h1 = jnp.maximum(_dot_ta(adj, p1), 0.0)
        mean1 = _dot(adj, _b(_dot(h1, wm1_ref[...])))
        std1 = jnp.exp(_dot(adj, _b(_dot(h1, wls1_ref[...]))))
        z1b = n3_ref[...] * std1 + mean1
        n1 = z1b.shape[0]
        zb_ref[0, :n1, :] = _b(n1_ref[...] * std1 + mean1)
        z1b_scr[...] = z1b
        rows = [z1b_scr[pl.ds(sidx_ref[j], 1), :]
                for j in range(n_species)]
        z3 = jnp.concatenate(rows, axis=0)
        zb_ref[0, n1:, :] = _b(z3)

    @pl.when(i == 1)
    def _encoder2():
        adj = adj_ref[...]
        p2 = _dot(x2_ref[...], wb2_ref[...])
        h2 = jnp.maximum(_dot(adj, p2), 0.0)
        mean2 = _dot_ta(adj, _b(_dot(h2, wm2_ref[...])))
        std2 = jnp.exp(_dot_ta(adj, _b(_dot(h2, wls2_ref[...]))))
        z2s = _b((n2_ref[...] * std2 + mean2)
                 * _signs(latent, grdpg, jnp.float32))
        zb_ref[0, :z2s.shape[0], :] = z2s


def _dec_kernel(z1t_ref, z2s_ref, z3b_ref,
                a_ref, a2_ref, z1o_ref, z2o_ref, z3_ref,
                *, latent, grdpg, sp_blk, half):
    i = pl.program_id(0)
    sg16 = _signs(latent, grdpg, jnp.bfloat16)
    z2s = z2s_ref[0]
    z1t = z1t_ref[0]
    z3h = z3b_ref[0, pl.ds(i * sp_blk, sp_blk), :]

    a_ref[...] = jax.nn.sigmoid(_dot_tb(z1t, z2s))
    a2_ref[...] = jax.nn.sigmoid(_dot_tb(z3h, z2s))
    z1o_ref[...] = z1t.astype(jnp.float32)
    z2o_ref[...] = (z2s_ref[0, pl.ds(i * half, half), :]
                    * sg16).astype(jnp.float32)
    z3_ref[...] = z3h.astype(jnp.float32)


@jax.jit
def kernel(w_base1, w_mean1, w_logstd1, w_base2, w_mean2, w_logstd2,
           X1, X2, adj, noise1, noise2, noise3, species_idx):
    N1, N2 = adj.shape
    latent = w_mean1.shape[1]
    n_species = species_idx.shape[0]

    enc_in = (X1, X2, adj, w_base1, w_mean1, w_logstd1,
              w_base2, w_mean2, w_logstd2, noise1, noise2, noise3)
    whole = [pl.BlockSpec(a.shape, lambda i, _sp, _n=a.ndim: (0,) * _n)
             for a in enc_in]

    zb = pl.pallas_call(
        functools.partial(_enc_kernel, latent=int(latent), grdpg=1,
                          n_species=int(n_species)),
        grid_spec=pltpu.PrefetchScalarGridSpec(
            num_scalar_prefetch=1,
            grid=(2,),
            in_specs=whole,
            out_specs=pl.BlockSpec((1, N1 + n_species, latent),
                                   lambda i, _sp: (i, 0, 0)),
            scratch_shapes=[pltpu.VMEM((N1, latent), jnp.float32)],
        ),
        out_shape=jax.ShapeDtypeStruct((2, N1 + n_species, latent),
                                       jnp.bfloat16),
        compiler_params=pltpu.CompilerParams(
            dimension_semantics=("parallel",)),
    )(species_idx, *enc_in)

    half = N1 // 2
    sp_blk = n_species // 2

    out_shapes = (
        jax.ShapeDtypeStruct((N1, N2), jnp.float32),
        jax.ShapeDtypeStruct((n_species, N2), jnp.float32),
        jax.ShapeDtypeStruct((N1, latent), jnp.float32),
        jax.ShapeDtypeStruct((N2, latent), jnp.float32),
        jax.ShapeDtypeStruct((n_species, latent), jnp.float32),
    )
    a_pred, a_pred2, z1o, z2o, z3 = pl.pallas_call(
        functools.partial(_dec_kernel, latent=int(latent), grdpg=1,
                          sp_blk=sp_blk, half=half),
        grid=(2,),
        in_specs=[
            pl.BlockSpec((1, half, latent), lambda i: (0, i, 0)),
            pl.BlockSpec((1, N2, latent), lambda i: (1, 0, 0)),
            pl.BlockSpec((1, n_species, latent),
                         lambda i, _r=N1 // n_species: (0, _r, 0)),
        ],
        out_specs=[
            pl.BlockSpec((half, N2), lambda i: (i, 0)),
            pl.BlockSpec((sp_blk, N2), lambda i: (i, 0)),
            pl.BlockSpec((half, latent), lambda i: (i, 0)),
            pl.BlockSpec((half, latent), lambda i: (i, 0)),
            pl.BlockSpec((sp_blk, latent), lambda i: (i, 0)),
        ],
        out_shape=out_shapes,
        compiler_params=pltpu.CompilerParams(
            dimension_semantics=("parallel",)),
    )(zb, zb, zb)

    return (a_pred, a_pred2, z1o, z2o, z3)
